# Initial kernel scaffold; baseline (speedup 1.0000x reference)
import jax
import jax.numpy as jnp
from jax.experimental import pallas as pl


def kernel(h, e, edge_index, snorm_n, snorm_e, emb_h_w, emb_h_b, emb_e_w, emb_e_b, A_w, A_b, B_w, B_b, C_w, C_b, D_w, D_b, E_w, E_b, bn_h_g, bn_h_b, bn_e_g, bn_e_b, mlp_w0, mlp_b0, mlp_w1, mlp_b1, mlp_w2, mlp_b2):
    raise NotImplementedError("write your pallas kernel here")


# speedup vs baseline: 1.1320x; 1.1320x over previous
import jax
import jax.numpy as jnp
from jax import lax
from jax.experimental import pallas as pl
from jax.experimental.pallas import tpu as pltpu
from jax.experimental.pallas import tpu_sc as plsc

_N = 10000
_E = 320000
_F = 128
_NL = 4
_NCORES = 2
_NSUB = 16
_NW = _NCORES * _NSUB
_EPW = _E // _NW
_C = 40
_NCHUNK = _EPW // _C
_ACCR = 10112
_TROW = _ACCR // _NSUB
_BE = 2000

_f32 = jnp.float32


def _embed_h_body(h_ref, w_ref, b_ref, o_ref):
    o_ref[...] = (
        jnp.dot(h_ref[...], w_ref[...], preferred_element_type=_f32)
        + b_ref[...]
    )


def _tables_body(h_ref, wa, ba, wd, bd, wb, bb, we, be,
                 ah_ref, td_ref, tb_ref, te_ref):
    x = h_ref[...]
    ah_ref[...] = jnp.dot(x, wa[...], preferred_element_type=_f32) + ba[...]
    td_ref[...] = jnp.dot(x, wd[...], preferred_element_type=_f32) + bd[...]
    tb_ref[...] = jnp.dot(x, wb[...], preferred_element_type=_f32) + bb[...]
    te_ref[...] = jnp.dot(x, we[...], preferred_element_type=_f32) + be[...]


def _hupd_body(ah_ref, hin_ref, snb_ref, bnp_ref, p_ref, out_ref):
    p = p_ref[...]
    num = p[0, 0, 0:_N, :] + p[1, 0, 0:_N, :]
    den = p[0, 1, 0:_N, :] + p[1, 1, 0:_N, :]
    m = num / (den + 1e-6)
    t = (ah_ref[...] + m) * snb_ref[...]
    mu = jnp.mean(t, axis=0, keepdims=True)
    var = jnp.mean(t * t, axis=0, keepdims=True) - mu * mu
    bp = bnp_ref[...]
    hb = bp[0:1] * (t - mu) * lax.rsqrt(var + 1e-5) + bp[1:2]
    out_ref[...] = hin_ref[...] + jnp.maximum(hb, 0.0)


def _mlp_body(h_ref, w0, b0, w1, b1, w2, b2, o_ref):
    y = jnp.maximum(
        jnp.dot(h_ref[...], w0[...], preferred_element_type=_f32) + b0[...],
        0.0)
    y = jnp.maximum(
        jnp.dot(y, w1[...], preferred_element_type=_f32) + b1[...], 0.0)
    o_ref[...] = jnp.dot(y, w2[...], preferred_element_type=_f32) + b2[...]


def _embed_e_body(e_ref, we, be, cw, cb, e0_ref, ce_ref):
    e0 = jnp.dot(e_ref[...], we[...], preferred_element_type=_f32) + be[...]
    e0_ref[...] = e0
    ce_ref[...] = jnp.dot(e0, cw[...], preferred_element_type=_f32) + cb[...]


def _apply_body(x_ref, ep_ref, st_ref, bnp_ref, cw_ref, cb_ref,
                en_ref, ce_ref):
    st = st_ref[...]
    ssum = jnp.sum(st[:, 0, :], axis=0, keepdims=True)
    ssq = jnp.sum(st[:, 1, :], axis=0, keepdims=True)
    mu = ssum * (1.0 / _E)
    var = ssq * (1.0 / _E) - mu * mu
    bp = bnp_ref[...]
    xb = bp[0:1] * (x_ref[...] - mu) * lax.rsqrt(var + 1e-5) + bp[1:2]
    en = ep_ref[...] + jnp.maximum(xb, 0.0)
    en_ref[...] = en
    ce_ref[...] = (
        jnp.dot(en, cw_ref[...], preferred_element_type=_f32) + cb_ref[...]
    )


def _full_call(body, n_out):
    return pl.pallas_call(
        body,
        out_shape=[jax.ShapeDtypeStruct((_N, _F), _f32)] * n_out,
    )


def _edge_blocked_call(body, n_out):
    nb = _E // _BE
    blk = pl.BlockSpec((_BE, _F), lambda i: (i, 0))
    wspec = pl.BlockSpec((_F, _F), lambda i: (0, 0))
    bspec = pl.BlockSpec((1, _F), lambda i: (0, 0))
    if body is _embed_e_body:
        in_specs = [blk, wspec, bspec, wspec, bspec]
    else:
        stspec = pl.BlockSpec((_NW, 8, _F), lambda i: (0, 0, 0))
        bnspec = pl.BlockSpec((2, _F), lambda i: (0, 0))
        in_specs = [blk, blk, stspec, bnspec, wspec, bspec]
    return pl.pallas_call(
        body,
        grid=(nb,),
        in_specs=in_specs,
        out_specs=[blk] * n_out,
        out_shape=[jax.ShapeDtypeStruct((_E, _F), _f32)] * n_out,
    )


_MESH = dict(core_axis_name="c", subcore_axis_name="s",
             num_cores=_NCORES, num_subcores=_NSUB)


def _edge_sc(with_x):
    out_type = [jax.ShapeDtypeStruct((_NCORES, 2, _ACCR, _F), _f32),
                jax.ShapeDtypeStruct((_E, _F), _f32)]
    if with_x:
        out_type += [jax.ShapeDtypeStruct((_E, _F), _f32),
                     jax.ShapeDtypeStruct((_NW, 8, _F), _f32)]
    scratch = [
        pltpu.VMEM_SHARED((_ACCR, _F), _f32),
        pltpu.VMEM((_C,), jnp.int32),
        pltpu.VMEM((_C,), jnp.int32),
        pltpu.VMEM((_C,), _f32),
        pltpu.VMEM((_C, _F), _f32),
        pltpu.VMEM((_C, _F), _f32),
        pltpu.VMEM((_C, _F), _f32),
        pltpu.VMEM((_C, _F), _f32),
        pltpu.VMEM((8, _F), _f32),
        pltpu.SemaphoreType.DMA,
    ]

    def body(td_h, tb_h, te_h, ce_h, src_h, dst_h, se_h, z_h, *refs):
        if with_x:
            p_h, sig_h, x_h, st_h = refs[0:4]
            rest = refs[4:]
        else:
            p_h, sig_h = refs[0:2]
            rest = refs[2:]
        (acc, src_b, dst_b, se_b, td_b, tb_b, te_b, ce_b, st_b,
         sem) = rest
        c = lax.axis_index("c")
        s = lax.axis_index("s")
        w = c * _NSUB + s
        trow = pl.ds(s * _TROW, _TROW)
        pltpu.sync_copy(z_h.at[trow], acc.at[trow])
        plsc.subcore_barrier()

        base = w * _EPW
        zv = jnp.zeros((16,), _f32)
        for v in range(8):
            st_b[0, pl.ds(v * 16, 16)] = zv
            st_b[1, pl.ds(v * 16, 16)] = zv

        def chunk_fn(i, carry):
            off = base + i * _C
            pltpu.sync_copy(src_h.at[pl.ds(off, _C)], src_b)
            pltpu.sync_copy(dst_h.at[pl.ds(off, _C)], dst_b)
            if with_x:
                pltpu.sync_copy(se_h.at[pl.ds(off, _C)], se_b)
            pltpu.sync_copy(ce_h.at[pl.ds(off, _C)], ce_b)
            pltpu.async_copy(td_h.at[src_b], td_b, sem).wait()
            pltpu.async_copy(tb_h.at[src_b], tb_b, sem).wait()
            pltpu.async_copy(te_h.at[dst_b], te_b, sem).wait()

            def row_fn(r, rc):
                if with_x:
                    grp = (r // 16) * 16
                    sv = se_b[pl.ds(grp, 16)]
                    sev = jnp.take_along_axis(
                        sv, jnp.full((16,), r - grp, jnp.int32), axis=0)
                for v in range(8):
                    sl = pl.ds(v * 16, 16)
                    en = td_b[r, sl] + te_b[r, sl] + ce_b[r, sl]
                    sg = 1.0 / (1.0 + jnp.exp(-en))
                    td_b[r, sl] = sg * tb_b[r, sl]
                    te_b[r, sl] = sg
                    if with_x:
                        x = en * sev
                        ce_b[r, sl] = x
                        if v < 5:
                            st_b[0, sl] = st_b[0, sl] + x
                            st_b[1, sl] = st_b[1, sl] + x * x
                return rc

            lax.fori_loop(0, _C, row_fn, 0)
            pltpu.sync_copy(td_b, acc.at[dst_b], add=True)
            pltpu.sync_copy(te_b, sig_h.at[pl.ds(off, _C)])
            if with_x:
                pltpu.sync_copy(ce_b, x_h.at[pl.ds(off, _C)])
            return carry

        lax.fori_loop(0, _NCHUNK, chunk_fn, 0)
        plsc.subcore_barrier()
        pltpu.sync_copy(acc.at[trow], p_h.at[c, 0, trow])
        pltpu.sync_copy(z_h.at[trow], acc.at[trow])
        if with_x:
            for v in range(8):
                for rr in range(2, 8):
                    st_b[rr, pl.ds(v * 16, 16)] = zv
            pltpu.sync_copy(st_b, st_h.at[w])
        plsc.subcore_barrier()

        def chunk2_fn(i, carry):
            off = base + i * _C
            pltpu.sync_copy(dst_h.at[pl.ds(off, _C)], dst_b)
            pltpu.sync_copy(sig_h.at[pl.ds(off, _C)], te_b)
            pltpu.sync_copy(te_b, acc.at[dst_b], add=True)
            return carry

        lax.fori_loop(0, _NCHUNK, chunk2_fn, 0)
        plsc.subcore_barrier()
        pltpu.sync_copy(acc.at[trow], p_h.at[c, 1, trow])

    return pl.kernel(
        body,
        out_type=out_type,
        mesh=plsc.VectorSubcoreMesh(**_MESH),
        scratch_types=scratch,
    )


def _pad2(wt):
    out = jnp.zeros((_F, _F), _f32)
    return out.at[: wt.shape[0], : wt.shape[1]].set(wt)


def _pad1(b):
    out = jnp.zeros((1, _F), _f32)
    return out.at[0, : b.shape[0]].set(b)


def _bnp(g, b):
    out = jnp.zeros((2, _F), _f32)
    return out.at[0, : g.shape[0]].set(g).at[1, : b.shape[0]].set(b)


def kernel(h, e, edge_index, snorm_n, snorm_e, emb_h_w, emb_h_b, emb_e_w,
           emb_e_b, A_w, A_b, B_w, B_b, C_w, C_b, D_w, D_b, E_w, E_b,
           bn_h_g, bn_h_b, bn_e_g, bn_e_b, mlp_w0, mlp_b0, mlp_w1, mlp_b1,
           mlp_w2, mlp_b2):
    src = edge_index[0]
    dst = edge_index[1]
    snb = jnp.broadcast_to(snorm_n[:, None], (_N, _F))
    zeros_acc = jnp.zeros((_ACCR, _F), _f32)

    embed_h = _full_call(_embed_h_body, 1)
    tables = _full_call(_tables_body, 4)
    hupd = _full_call(_hupd_body, 1)
    mlp = _full_call(_mlp_body, 1)
    embed_e = _edge_blocked_call(_embed_e_body, 2)
    apply_e = _edge_blocked_call(_apply_body, 2)
    edge_full = _edge_sc(True)
    edge_last = _edge_sc(False)

    (hcur,) = embed_h(h, _pad2(emb_h_w), _pad1(emb_h_b))
    ecur, ce = embed_e(e, _pad2(emb_e_w), _pad1(emb_e_b),
                       _pad2(C_w[0]), _pad1(C_b[0]))

    for l in range(_NL):
        ah, td, tb, te = tables(
            hcur,
            _pad2(A_w[l]), _pad1(A_b[l]), _pad2(D_w[l]), _pad1(D_b[l]),
            _pad2(B_w[l]), _pad1(B_b[l]), _pad2(E_w[l]), _pad1(E_b[l]))
        if l < _NL - 1:
            p, _sig, x, st = edge_full(td, tb, te, ce, src, dst, snorm_e,
                                       zeros_acc)
        else:
            p, _sig = edge_last(td, tb, te, ce, src, dst, snorm_e,
                                zeros_acc)
        (hcur,) = hupd(ah, hcur, snb, _bnp(bn_h_g[l], bn_h_b[l]), p)
        if l < _NL - 1:
            ecur, ce = apply_e(x, ecur, st, _bnp(bn_e_g[l], bn_e_b[l]),
                               _pad2(C_w[l + 1]), _pad1(C_b[l + 1]))

    (y,) = mlp(hcur, _pad2(mlp_w0), _pad1(mlp_b0), _pad2(mlp_w1),
               _pad1(mlp_b1), _pad2(mlp_w2), _pad1(mlp_b2))
    return y[:, : mlp_w2.shape[1]]


# speedup vs baseline: 3.5999x; 3.1802x over previous
import jax
import jax.numpy as jnp
from jax import lax
from jax.experimental import pallas as pl
from jax.experimental.pallas import tpu as pltpu
from jax.experimental.pallas import tpu_sc as plsc

_N = 10000
_E = 320000
_F = 128
_NL = 4
_NCORES = 2
_NSUB = 16
_NW = _NCORES * _NSUB
_EPW = _E // _NW
_C = 80
_NCHUNK = _EPW // _C
_C2 = 200
_NCH2 = _EPW // _C2
_ACCR = 10112
_TROW = _ACCR // _NSUB
_BE = 2000
_BES = 1280

_f32 = jnp.float32


def _embed_h_body(h_ref, w_ref, b_ref, o_ref):
    o_ref[...] = (
        jnp.dot(h_ref[...], w_ref[...], preferred_element_type=_f32)
        + b_ref[...]
    )


def _tables_body(h_ref, wa, ba, wd, bd, wb, bb, we, be,
                 ah_ref, td_ref, tb_ref, te_ref):
    x = h_ref[...]
    ah_ref[...] = jnp.dot(x, wa[...], preferred_element_type=_f32) + ba[...]
    td_ref[...] = jnp.dot(x, wd[...], preferred_element_type=_f32) + bd[...]
    tb_ref[...] = jnp.dot(x, wb[...], preferred_element_type=_f32) + bb[...]
    te_ref[...] = jnp.dot(x, we[...], preferred_element_type=_f32) + be[...]


def _hupd_body(ah_ref, hin_ref, snb_ref, bnp_ref, np_ref, dp_ref, out_ref):
    nump = np_ref[...]
    denp = dp_ref[...]
    num = nump[0, 0:_N, :] + nump[1, 0:_N, :]
    den = denp[0, 0:_N, :] + denp[1, 0:_N, :]
    m = num / (den + 1e-6)
    t = (ah_ref[...] + m) * snb_ref[...]
    mu = jnp.mean(t, axis=0, keepdims=True)
    var = jnp.mean(t * t, axis=0, keepdims=True) - mu * mu
    bp = bnp_ref[...]
    hb = bp[0:1] * (t - mu) * lax.rsqrt(var + 1e-5) + bp[1:2]
    out_ref[...] = hin_ref[...] + jnp.maximum(hb, 0.0)


def _mlp_body(h_ref, w0, b0, w1, b1, w2, b2, o_ref):
    y = jnp.maximum(
        jnp.dot(h_ref[...], w0[...], preferred_element_type=_f32) + b0[...],
        0.0)
    y = jnp.maximum(
        jnp.dot(y, w1[...], preferred_element_type=_f32) + b1[...], 0.0)
    o_ref[...] = jnp.dot(y, w2[...], preferred_element_type=_f32) + b2[...]


def _embed_e_body(e_ref, we, be, cw, cb, e0_ref, ce_ref):
    e0 = jnp.dot(e_ref[...], we[...], preferred_element_type=_f32) + be[...]
    e0_ref[...] = e0
    ce_ref[...] = jnp.dot(e0, cw[...], preferred_element_type=_f32) + cb[...]


def _row_bcast(sb):
    parts = []
    for j in range(_BES // _F):
        t = jnp.broadcast_to(sb[j:j + 1, :], (_F, _F))
        parts.append(t.T)
    return jnp.concatenate(parts, axis=0)


def _estats_body(en_ref, se_ref, st_ref):
    i = pl.program_id(0)
    x = en_ref[...] * _row_bcast(se_ref[0])
    upd = jnp.concatenate(
        [jnp.sum(x, axis=0, keepdims=True),
         jnp.sum(x * x, axis=0, keepdims=True),
         jnp.zeros((6, _F), _f32)], axis=0)

    @pl.when(i == 0)
    def _init():
        st_ref[...] = upd

    @pl.when(i > 0)
    def _acc():
        st_ref[...] = st_ref[...] + upd


def _apply_body(en_ref, ep_ref, se_ref, st_ref, bnp_ref, cw_ref, cb_ref,
                eo_ref, ce_ref):
    st = st_ref[...]
    mu = st[0:1] * (1.0 / _E)
    var = st[1:2] * (1.0 / _E) - mu * mu
    x = en_ref[...] * _row_bcast(se_ref[0])
    bp = bnp_ref[...]
    xb = bp[0:1] * (x - mu) * lax.rsqrt(var + 1e-5) + bp[1:2]
    eo = ep_ref[...] + jnp.maximum(xb, 0.0)
    eo_ref[...] = eo
    ce_ref[...] = (
        jnp.dot(eo, cw_ref[...], preferred_element_type=_f32) + cb_ref[...]
    )


def _full_call(body, n_out):
    return pl.pallas_call(
        body,
        out_shape=[jax.ShapeDtypeStruct((_N, _F), _f32)] * n_out,
    )


def _embed_e_call():
    nb = _E // _BE
    blk = pl.BlockSpec((_BE, _F), lambda i: (i, 0))
    wspec = pl.BlockSpec((_F, _F), lambda i: (0, 0))
    bspec = pl.BlockSpec((1, _F), lambda i: (0, 0))
    return pl.pallas_call(
        _embed_e_body,
        grid=(nb,),
        in_specs=[blk, wspec, bspec, wspec, bspec],
        out_specs=[blk, blk],
        out_shape=[jax.ShapeDtypeStruct((_E, _F), _f32)] * 2,
    )


def _estats_call():
    nb = _E // _BES
    blk = pl.BlockSpec((_BES, _F), lambda i: (i, 0))
    sespec = pl.BlockSpec((1, _BES // _F, _F), lambda i: (i, 0, 0))
    stspec = pl.BlockSpec((8, _F), lambda i: (0, 0))
    return pl.pallas_call(
        _estats_body,
        grid=(nb,),
        in_specs=[blk, sespec],
        out_specs=stspec,
        out_shape=jax.ShapeDtypeStruct((8, _F), _f32),
    )


def _apply_call():
    nb = _E // _BES
    blk = pl.BlockSpec((_BES, _F), lambda i: (i, 0))
    sespec = pl.BlockSpec((1, _BES // _F, _F), lambda i: (i, 0, 0))
    stspec = pl.BlockSpec((8, _F), lambda i: (0, 0))
    wspec = pl.BlockSpec((_F, _F), lambda i: (0, 0))
    bspec = pl.BlockSpec((1, _F), lambda i: (0, 0))
    bnspec = pl.BlockSpec((2, _F), lambda i: (0, 0))
    return pl.pallas_call(
        _apply_body,
        grid=(nb,),
        in_specs=[blk, blk, sespec, stspec, bnspec, wspec, bspec],
        out_specs=[blk, blk],
        out_shape=[jax.ShapeDtypeStruct((_E, _F), _f32)] * 2,
    )


_MESH = dict(core_axis_name="c", subcore_axis_name="s",
             num_cores=_NCORES, num_subcores=_NSUB)


def _phase1_sc(spill_en):
    out_type = [jax.ShapeDtypeStruct((_NCORES, _ACCR, _F), _f32),
                jax.ShapeDtypeStruct((_E, _F), _f32)]
    scratch = [
        pltpu.VMEM_SHARED((_ACCR, _F), _f32),
        pltpu.VMEM((_C,), jnp.int32),
        pltpu.VMEM((_C,), jnp.int32),
        pltpu.VMEM((_C, _F), _f32),
        pltpu.VMEM((_C, _F), _f32),
        pltpu.VMEM((_C, _F), _f32),
        pltpu.VMEM((_C, _F), _f32),
        pltpu.SemaphoreType.DMA,
        pltpu.SemaphoreType.DMA,
    ]

    def body(td_h, tb_h, te_h, ce_h, src_h, dst_h, z_h, nump_h, spill_h,
             acc, src_b, dst_b, td_b, tb_b, te_b, ce_b, sem, sem2):
        c = lax.axis_index("c")
        s = lax.axis_index("s")
        w = c * _NSUB + s
        trow = pl.ds(s * _TROW, _TROW)
        pltpu.sync_copy(z_h.at[trow], acc.at[trow])
        plsc.subcore_barrier()
        base = w * _EPW

        def chunk_fn(i, carry):
            off = base + i * _C
            d1 = pltpu.async_copy(src_h.at[pl.ds(off, _C)], src_b, sem)
            d2 = pltpu.async_copy(dst_h.at[pl.ds(off, _C)], dst_b, sem)
            d3 = pltpu.async_copy(ce_h.at[pl.ds(off, _C)], ce_b, sem2)
            d1.wait()
            d2.wait()
            g1 = pltpu.async_copy(td_h.at[src_b], td_b, sem)
            g2 = pltpu.async_copy(tb_h.at[src_b], tb_b, sem)
            g3 = pltpu.async_copy(te_h.at[dst_b], te_b, sem)
            d3.wait()
            g1.wait()
            g2.wait()
            g3.wait()

            def row_fn(r, rc):
                for v in range(8):
                    sl = pl.ds(v * 16, 16)
                    en = td_b[r, sl] + te_b[r, sl] + ce_b[r, sl]
                    sg = 1.0 / (1.0 + jnp.exp(-en))
                    td_b[r, sl] = sg * tb_b[r, sl]
                    ce_b[r, sl] = en if spill_en else sg
                return rc

            lax.fori_loop(0, _C, row_fn, 0)
            o1 = pltpu.async_copy(td_b, acc.at[dst_b], sem, add=True)
            o2 = pltpu.async_copy(ce_b, spill_h.at[pl.ds(off, _C)], sem2)
            o1.wait()
            o2.wait()
            return carry

        lax.fori_loop(0, _NCHUNK, chunk_fn, 0)
        plsc.subcore_barrier()
        pltpu.sync_copy(acc.at[trow], nump_h.at[c, trow])

    return pl.kernel(
        body,
        out_type=out_type,
        mesh=plsc.VectorSubcoreMesh(**_MESH),
        scratch_types=scratch,
    )


def _phase2_sc(recompute):
    out_type = [jax.ShapeDtypeStruct((_NCORES, _ACCR, _F), _f32)]
    scratch = [
        pltpu.VMEM_SHARED((_ACCR, _F), _f32),
        pltpu.VMEM((_C2,), jnp.int32),
        pltpu.VMEM((_C2, _F), _f32),
        pltpu.SemaphoreType.DMA,
    ]

    def body(spill_h, dst_h, z_h, denp_h, acc, dst_b, en_b, sem):
        c = lax.axis_index("c")
        s = lax.axis_index("s")
        w = c * _NSUB + s
        trow = pl.ds(s * _TROW, _TROW)
        pltpu.sync_copy(z_h.at[trow], acc.at[trow])
        plsc.subcore_barrier()
        base = w * _EPW

        def chunk_fn(i, carry):
            off = base + i * _C2
            d = pltpu.async_copy(spill_h.at[pl.ds(off, _C2)], en_b, sem)
            d1 = pltpu.async_copy(dst_h.at[pl.ds(off, _C2)], dst_b, sem)
            d.wait()
            d1.wait()
            if recompute:
                def row_fn(r, rc):
                    for v in range(8):
                        sl = pl.ds(v * 16, 16)
                        en_b[r, sl] = 1.0 / (1.0 + jnp.exp(-en_b[r, sl]))
                    return rc

                lax.fori_loop(0, _C2, row_fn, 0)
            o = pltpu.async_copy(en_b, acc.at[dst_b], sem, add=True)
            o.wait()
            return carry

        lax.fori_loop(0, _NCH2, chunk_fn, 0)
        plsc.subcore_barrier()
        pltpu.sync_copy(acc.at[trow], denp_h.at[c, trow])

    return pl.kernel(
        body,
        out_type=out_type,
        mesh=plsc.VectorSubcoreMesh(**_MESH),
        scratch_types=scratch,
    )


def _pad2(wt):
    out = jnp.zeros((_F, _F), _f32)
    return out.at[: wt.shape[0], : wt.shape[1]].set(wt)


def _pad1(b):
    out = jnp.zeros((1, _F), _f32)
    return out.at[0, : b.shape[0]].set(b)


def _bnp(g, b):
    out = jnp.zeros((2, _F), _f32)
    return out.at[0, : g.shape[0]].set(g).at[1, : b.shape[0]].set(b)


def kernel(h, e, edge_index, snorm_n, snorm_e, emb_h_w, emb_h_b, emb_e_w,
           emb_e_b, A_w, A_b, B_w, B_b, C_w, C_b, D_w, D_b, E_w, E_b,
           bn_h_g, bn_h_b, bn_e_g, bn_e_b, mlp_w0, mlp_b0, mlp_w1, mlp_b1,
           mlp_w2, mlp_b2):
    src = edge_index[0]
    dst = edge_index[1]
    se2 = snorm_e.reshape(_E // _BES, _BES // _F, _F)
    snb = jnp.broadcast_to(snorm_n[:, None], (_N, _F))
    zeros_acc = jnp.zeros((_ACCR, _F), _f32)

    embed_h = _full_call(_embed_h_body, 1)
    tables = _full_call(_tables_body, 4)
    hupd = _full_call(_hupd_body, 1)
    mlp = _full_call(_mlp_body, 1)
    embed_e = _embed_e_call()
    estats = _estats_call()
    apply_e = _apply_call()
    p1_en = _phase1_sc(True)
    p1_sig = _phase1_sc(False)
    p2_rec = _phase2_sc(True)
    p2_dir = _phase2_sc(False)

    (hcur,) = embed_h(h, _pad2(emb_h_w), _pad1(emb_h_b))
    ecur, ce = embed_e(e, _pad2(emb_e_w), _pad1(emb_e_b),
                       _pad2(C_w[0]), _pad1(C_b[0]))

    for l in range(_NL):
        ah, td, tb, te = tables(
            hcur,
            _pad2(A_w[l]), _pad1(A_b[l]), _pad2(D_w[l]), _pad1(D_b[l]),
            _pad2(B_w[l]), _pad1(B_b[l]), _pad2(E_w[l]), _pad1(E_b[l]))
        if l < _NL - 1:
            nump, spill = p1_en(td, tb, te, ce, src, dst, zeros_acc)
            denp = p2_rec(spill, dst, zeros_acc)
            if isinstance(denp, (tuple, list)):
                (denp,) = denp
            st = estats(spill, se2)
            ecur, ce = apply_e(spill, ecur, se2, st,
                               _bnp(bn_e_g[l], bn_e_b[l]),
                               _pad2(C_w[l + 1]), _pad1(C_b[l + 1]))
        else:
            nump, spill = p1_sig(td, tb, te, ce, src, dst, zeros_acc)
            denp = p2_dir(spill, dst, zeros_acc)
            if isinstance(denp, (tuple, list)):
                (denp,) = denp
        (hcur,) = hupd(ah, hcur, snb, _bnp(bn_h_g[l], bn_h_b[l]),
                       nump, denp)

    (y,) = mlp(hcur, _pad2(mlp_w0), _pad1(mlp_b0), _pad2(mlp_w1),
               _pad1(mlp_b1), _pad2(mlp_w2), _pad1(mlp_b2))
    return y[:, : mlp_w2.shape[1]]


# speedup vs baseline: 3.6008x; 1.0002x over previous
import jax
import jax.numpy as jnp
from jax import lax
from jax.experimental import pallas as pl
from jax.experimental.pallas import tpu as pltpu
from jax.experimental.pallas import tpu_sc as plsc

_N = 10000
_E = 320000
_F = 128
_NL = 4
_NCORES = 2
_NSUB = 16
_NW = _NCORES * _NSUB
_EPW = _E // _NW
_C = 80
_NCHUNK = _EPW // _C
_C2 = 200
_NCH2 = _EPW // _C2
_ACCR = 10112
_TROW = _ACCR // _NSUB
_BE = 2000
_BES = 1280

_f32 = jnp.float32


def _embed_h_body(h_ref, w_ref, b_ref, o_ref):
    o_ref[...] = (
        jnp.dot(h_ref[...], w_ref[...], preferred_element_type=_f32)
        + b_ref[...]
    )


def _tables_body(h_ref, wa, ba, wd, bd, wb, bb, we, be,
                 ah_ref, td_ref, tb_ref, te_ref):
    x = h_ref[...]
    ah_ref[...] = jnp.dot(x, wa[...], preferred_element_type=_f32) + ba[...]
    td_ref[...] = jnp.dot(x, wd[...], preferred_element_type=_f32) + bd[...]
    tb_ref[...] = jnp.dot(x, wb[...], preferred_element_type=_f32) + bb[...]
    te_ref[...] = jnp.dot(x, we[...], preferred_element_type=_f32) + be[...]


def _hupd_body(ah_ref, hin_ref, snb_ref, bnp_ref, np_ref, dp_ref, out_ref):
    nump = np_ref[...]
    denp = dp_ref[...]
    num = nump[0, 0:_N, :] + nump[1, 0:_N, :]
    den = denp[0, 0:_N, :] + denp[1, 0:_N, :]
    m = num / (den + 1e-6)
    t = (ah_ref[...] + m) * snb_ref[...]
    mu = jnp.mean(t, axis=0, keepdims=True)
    var = jnp.mean(t * t, axis=0, keepdims=True) - mu * mu
    bp = bnp_ref[...]
    hb = bp[0:1] * (t - mu) * lax.rsqrt(var + 1e-5) + bp[1:2]
    out_ref[...] = hin_ref[...] + jnp.maximum(hb, 0.0)


def _mlp_body(h_ref, w0, b0, w1, b1, w2, b2, o_ref):
    y = jnp.maximum(
        jnp.dot(h_ref[...], w0[...], preferred_element_type=_f32) + b0[...],
        0.0)
    y = jnp.maximum(
        jnp.dot(y, w1[...], preferred_element_type=_f32) + b1[...], 0.0)
    o_ref[...] = jnp.dot(y, w2[...], preferred_element_type=_f32) + b2[...]


def _embed_e_body(e_ref, we, be, cw, cb, e0_ref, ce_ref):
    e0 = jnp.dot(e_ref[...], we[...], preferred_element_type=_f32) + be[...]
    e0_ref[...] = e0
    ce_ref[...] = jnp.dot(e0, cw[...], preferred_element_type=_f32) + cb[...]


def _row_bcast(sb):
    parts = []
    for j in range(_BES // _F):
        t = jnp.broadcast_to(sb[j:j + 1, :], (_F, _F))
        parts.append(t.T)
    return jnp.concatenate(parts, axis=0)


def _estats_body(en_ref, se_ref, st_ref):
    i = pl.program_id(0)
    x = en_ref[...] * _row_bcast(se_ref[0])
    upd = jnp.concatenate(
        [jnp.sum(x, axis=0, keepdims=True),
         jnp.sum(x * x, axis=0, keepdims=True),
         jnp.zeros((6, _F), _f32)], axis=0)

    @pl.when(i == 0)
    def _init():
        st_ref[...] = upd

    @pl.when(i > 0)
    def _acc():
        st_ref[...] = st_ref[...] + upd


def _apply_body(en_ref, ep_ref, se_ref, st_ref, bnp_ref, cw_ref, cb_ref,
                eo_ref, ce_ref):
    st = st_ref[...]
    mu = st[0:1] * (1.0 / _E)
    var = st[1:2] * (1.0 / _E) - mu * mu
    x = en_ref[...] * _row_bcast(se_ref[0])
    bp = bnp_ref[...]
    xb = bp[0:1] * (x - mu) * lax.rsqrt(var + 1e-5) + bp[1:2]
    eo = ep_ref[...] + jnp.maximum(xb, 0.0)
    eo_ref[...] = eo
    ce_ref[...] = (
        jnp.dot(eo, cw_ref[...], preferred_element_type=_f32) + cb_ref[...]
    )


def _full_call(body, n_out):
    return pl.pallas_call(
        body,
        out_shape=[jax.ShapeDtypeStruct((_N, _F), _f32)] * n_out,
    )


def _embed_e_call():
    nb = _E // _BE
    blk = pl.BlockSpec((_BE, _F), lambda i: (i, 0))
    wspec = pl.BlockSpec((_F, _F), lambda i: (0, 0))
    bspec = pl.BlockSpec((1, _F), lambda i: (0, 0))
    return pl.pallas_call(
        _embed_e_body,
        grid=(nb,),
        in_specs=[blk, wspec, bspec, wspec, bspec],
        out_specs=[blk, blk],
        out_shape=[jax.ShapeDtypeStruct((_E, _F), _f32)] * 2,
    )


def _estats_call():
    nb = _E // _BES
    blk = pl.BlockSpec((_BES, _F), lambda i: (i, 0))
    sespec = pl.BlockSpec((1, _BES // _F, _F), lambda i: (i, 0, 0))
    stspec = pl.BlockSpec((8, _F), lambda i: (0, 0))
    return pl.pallas_call(
        _estats_body,
        grid=(nb,),
        in_specs=[blk, sespec],
        out_specs=stspec,
        out_shape=jax.ShapeDtypeStruct((8, _F), _f32),
    )


def _apply_call():
    nb = _E // _BES
    blk = pl.BlockSpec((_BES, _F), lambda i: (i, 0))
    sespec = pl.BlockSpec((1, _BES // _F, _F), lambda i: (i, 0, 0))
    stspec = pl.BlockSpec((8, _F), lambda i: (0, 0))
    wspec = pl.BlockSpec((_F, _F), lambda i: (0, 0))
    bspec = pl.BlockSpec((1, _F), lambda i: (0, 0))
    bnspec = pl.BlockSpec((2, _F), lambda i: (0, 0))
    return pl.pallas_call(
        _apply_body,
        grid=(nb,),
        in_specs=[blk, blk, sespec, stspec, bnspec, wspec, bspec],
        out_specs=[blk, blk],
        out_shape=[jax.ShapeDtypeStruct((_E, _F), _f32)] * 2,
    )


_MESH = dict(core_axis_name="c", subcore_axis_name="s",
             num_cores=_NCORES, num_subcores=_NSUB)


def _phase1_sc(spill_en):
    out_type = [jax.ShapeDtypeStruct((_NCORES, _ACCR, _F), _f32),
                jax.ShapeDtypeStruct((_E, _F), _f32)]
    scratch = [
        pltpu.VMEM_SHARED((_ACCR, _F), _f32),
        pltpu.VMEM((_C,), jnp.int32),
        pltpu.VMEM((_C,), jnp.int32),
        pltpu.VMEM((_C, _F), _f32),
        pltpu.VMEM((_C, _F), _f32),
        pltpu.VMEM((_C, _F), _f32),
        pltpu.VMEM((_C, _F), _f32),
        pltpu.SemaphoreType.DMA,
        pltpu.SemaphoreType.DMA,
    ]

    def body(td_h, tb_h, te_h, ce_h, src_h, dst_h, z_h, nump_h, spill_h,
             acc, src_b, dst_b, td_b, tb_b, te_b, ce_b, sem, sem2):
        c = lax.axis_index("c")
        s = lax.axis_index("s")
        w = c * _NSUB + s
        trow = pl.ds(s * _TROW, _TROW)
        pltpu.sync_copy(z_h.at[trow], acc.at[trow])
        plsc.subcore_barrier()
        base = w * _EPW

        def chunk_fn(i, carry):
            off = base + i * _C
            d1 = pltpu.async_copy(src_h.at[pl.ds(off, _C)], src_b, sem)
            d2 = pltpu.async_copy(dst_h.at[pl.ds(off, _C)], dst_b, sem)
            d3 = pltpu.async_copy(ce_h.at[pl.ds(off, _C)], ce_b, sem2)
            d1.wait()
            d2.wait()
            g1 = pltpu.async_copy(td_h.at[src_b], td_b, sem)
            g2 = pltpu.async_copy(tb_h.at[src_b], tb_b, sem)
            g3 = pltpu.async_copy(te_h.at[dst_b], te_b, sem)
            d3.wait()
            g1.wait()
            g2.wait()
            g3.wait()

            def row_fn(r, rc):
                for v in range(8):
                    sl = pl.ds(v * 16, 16)
                    en = td_b[r, sl] + te_b[r, sl] + ce_b[r, sl]
                    sg = 1.0 / (1.0 + jnp.exp(-en))
                    td_b[r, sl] = sg * tb_b[r, sl]
                    ce_b[r, sl] = en if spill_en else sg
                return rc

            lax.fori_loop(0, _C, row_fn, 0)
            o1 = pltpu.async_copy(td_b, acc.at[dst_b], sem, add=True)
            o2 = pltpu.async_copy(ce_b, spill_h.at[pl.ds(off, _C)], sem2)
            o1.wait()
            o2.wait()
            return carry

        lax.fori_loop(0, _NCHUNK, chunk_fn, 0)
        plsc.subcore_barrier()
        pltpu.sync_copy(acc.at[trow], nump_h.at[c, trow])

    return pl.kernel(
        body,
        out_type=out_type,
        mesh=plsc.VectorSubcoreMesh(**_MESH),
        scratch_types=scratch,
    )


def _phase2_sc(recompute):
    out_type = [jax.ShapeDtypeStruct((_NCORES, _ACCR, _F), _f32)]
    scratch = [
        pltpu.VMEM_SHARED((_ACCR, _F), _f32),
        pltpu.VMEM((_C2,), jnp.int32),
        pltpu.VMEM((_C2, _F), _f32),
        pltpu.SemaphoreType.DMA,
    ]

    def body(spill_h, dst_h, z_h, denp_h, acc, dst_b, en_b, sem):
        c = lax.axis_index("c")
        s = lax.axis_index("s")
        w = c * _NSUB + s
        trow = pl.ds(s * _TROW, _TROW)
        pltpu.sync_copy(z_h.at[trow], acc.at[trow])
        plsc.subcore_barrier()
        base = w * _EPW

        def chunk_fn(i, carry):
            off = base + i * _C2
            d = pltpu.async_copy(spill_h.at[pl.ds(off, _C2)], en_b, sem)
            d1 = pltpu.async_copy(dst_h.at[pl.ds(off, _C2)], dst_b, sem)
            d.wait()
            d1.wait()
            if recompute:
                def row_fn(r, rc):
                    for v in range(8):
                        sl = pl.ds(v * 16, 16)
                        en_b[r, sl] = 1.0 / (1.0 + jnp.exp(-en_b[r, sl]))
                    return rc

                lax.fori_loop(0, _C2, row_fn, 0)
            o = pltpu.async_copy(en_b, acc.at[dst_b], sem, add=True)
            o.wait()
            return carry

        lax.fori_loop(0, _NCH2, chunk_fn, 0)
        plsc.subcore_barrier()
        pltpu.sync_copy(acc.at[trow], denp_h.at[c, trow])

    return pl.kernel(
        body,
        out_type=out_type,
        mesh=plsc.VectorSubcoreMesh(**_MESH),
        scratch_types=scratch,
    )


def _pad2(wt):
    out = jnp.zeros((_F, _F), _f32)
    return out.at[: wt.shape[0], : wt.shape[1]].set(wt)


def _pad1(b):
    out = jnp.zeros((1, _F), _f32)
    return out.at[0, : b.shape[0]].set(b)


def _bnp(g, b):
    out = jnp.zeros((2, _F), _f32)
    return out.at[0, : g.shape[0]].set(g).at[1, : b.shape[0]].set(b)


def kernel(h, e, edge_index, snorm_n, snorm_e, emb_h_w, emb_h_b, emb_e_w,
           emb_e_b, A_w, A_b, B_w, B_b, C_w, C_b, D_w, D_b, E_w, E_b,
           bn_h_g, bn_h_b, bn_e_g, bn_e_b, mlp_w0, mlp_b0, mlp_w1, mlp_b1,
           mlp_w2, mlp_b2):
    src = edge_index[0]
    dst = edge_index[1]
    se2 = snorm_e.reshape(_E // _BES, _BES // _F, _F)
    snb = jnp.broadcast_to(snorm_n[:, None], (_N, _F))
    zeros_acc = jnp.zeros((_ACCR, _F), _f32)

    embed_h = _full_call(_embed_h_body, 1)
    tables = _full_call(_tables_body, 4)
    hupd = _full_call(_hupd_body, 1)
    mlp = _full_call(_mlp_body, 1)
    embed_e = _embed_e_call()
    estats = _estats_call()
    apply_e = _apply_call()
    p1_en = _phase1_sc(True)
    p1_sig = _phase1_sc(False)
    p2_rec = _phase2_sc(True)
    p2_dir = _phase2_sc(False)

    (hcur,) = embed_h(h, _pad2(emb_h_w), _pad1(emb_h_b))
    ecur, ce = embed_e(e, _pad2(emb_e_w), _pad1(emb_e_b),
                       _pad2(C_w[0]), _pad1(C_b[0]))

    for l in range(_NL):
        ah, td, tb, te = tables(
            hcur,
            _pad2(A_w[l]), _pad1(A_b[l]), _pad2(D_w[l]), _pad1(D_b[l]),
            _pad2(B_w[l]), _pad1(B_b[l]), _pad2(E_w[l]), _pad1(E_b[l]))
        p1 = p1_en if l < _NL - 1 else p1_sig
        nump, spill = p1(td, tb, te, ce, src, dst, zeros_acc)
        p2 = p2_rec if l < _NL - 1 else p2_dir
        denp = p2(spill, dst, zeros_acc)
        if isinstance(denp, (tuple, list)):
            (denp,) = denp
        if l < _NL - 1:
            st = estats(spill, se2)
            ecur, ce = apply_e(spill, ecur, se2, st,
                               _bnp(bn_e_g[l], bn_e_b[l]),
                               _pad2(C_w[l + 1]), _pad1(C_b[l + 1]))
        (hcur,) = hupd(ah, hcur, snb, _bnp(bn_h_g[l], bn_h_b[l]),
                       nump, denp)

    (y,) = mlp(hcur, _pad2(mlp_w0), _pad1(mlp_b0), _pad2(mlp_w1),
               _pad1(mlp_b1), _pad2(mlp_w2), _pad1(mlp_b2))
    return y[:, : mlp_w2.shape[1]]
